# Initial kernel scaffold; baseline (speedup 1.0000x reference)
import jax
import jax.numpy as jnp
from jax.experimental import pallas as pl


def kernel(x, Wq, Wc, Wk, Wv, Wo):
    raise NotImplementedError("write your pallas kernel here")


# speedup vs baseline: 7.3748x; 7.3748x over previous
import functools

import jax
import jax.numpy as jnp
from jax.experimental import pallas as pl

_HID = 1024
_LAT = 512
_H = 16
_HD = 64
_W = 128
_P = 16
_FORGET = 0.1
_NEG = float(jnp.finfo(jnp.float32).min)
_BQ = 256

_keep_cache = {}


def _keep_mask(h, s):
    key = (h, s)
    if key not in _keep_cache:
        with jax.ensure_compile_time_eval():
            fkey = jax.random.key(42)
            r = jax.random.uniform(fkey, (1, h, s, s)) >= _FORGET
            _keep_cache[key] = r[0].astype(jnp.int8)
    return _keep_cache[key]


def _proj_body(x_ref, wq_ref, wc_ref, wk_ref, wv_ref, q_ref, k_ref, v_ref):
    x = x_ref[...]
    q_ref[...] = jnp.dot(x, wq_ref[...], preferred_element_type=jnp.float32)
    c = jnp.dot(x, wc_ref[...], preferred_element_type=jnp.float32)
    k_ref[...] = jnp.dot(c, wk_ref[...], preferred_element_type=jnp.float32)
    v_ref[...] = jnp.dot(c, wv_ref[...], preferred_element_type=jnp.float32)


def _attn_body(q_ref, k_ref, v_ref, rnd_ref, o_ref, *, bq, s, scale):
    qb = pl.program_id(1)
    q = q_ref[0]
    k = k_ref[0]
    sc = jax.lax.dot_general(q, k, (((1,), (1,)), ((), ())),
                             preferred_element_type=jnp.float32) * scale
    row = qb * bq + jax.lax.broadcasted_iota(jnp.int32, (bq, s), 0)
    col = jax.lax.broadcasted_iota(jnp.int32, (bq, s), 1)
    causal = col <= row
    scz = sc + 0.0
    bits = jax.lax.bitcast_convert_type(scz, jnp.uint32)
    u = jnp.where(scz >= 0, bits | jnp.uint32(0x80000000), ~bits)
    u = jnp.where(causal, u, jnp.uint32(0))

    def body(i, prefix):
        cand = prefix | (jnp.uint32(1) << (31 - i).astype(jnp.uint32))
        cnt = jnp.sum((u >= cand).astype(jnp.int32), axis=1, keepdims=True)
        return jnp.where(cnt >= _W, cand, prefix)

    kth = jax.lax.fori_loop(0, 32, body, jnp.zeros((bq, 1), jnp.uint32))
    topk = u >= kth
    keep = (topk & (rnd_ref[0] != 0)) | (col < _P) | (col == row)
    keep = keep & causal
    m = jnp.max(jnp.where(keep, sc, _NEG), axis=1, keepdims=True)
    p = jnp.where(keep, jnp.exp(sc - m), 0.0)
    denom = jnp.sum(p, axis=1, keepdims=True)
    o_ref[0] = jnp.dot(p / denom, v_ref[0],
                       preferred_element_type=jnp.float32)


def _outproj_body(a_ref, wo_ref, o_ref):
    o_ref[...] = jnp.dot(a_ref[...], wo_ref[...],
                         preferred_element_type=jnp.float32)


def kernel(x, Wq, Wc, Wk, Wv, Wo):
    b, s, d = x.shape
    x2 = x.reshape(s, d)
    q2, k2, v2 = pl.pallas_call(
        _proj_body,
        grid=(s // _BQ,),
        in_specs=[pl.BlockSpec((_BQ, d), lambda i: (i, 0)),
                  pl.BlockSpec((d, d), lambda i: (0, 0)),
                  pl.BlockSpec((d, _LAT), lambda i: (0, 0)),
                  pl.BlockSpec((_LAT, d), lambda i: (0, 0)),
                  pl.BlockSpec((_LAT, d), lambda i: (0, 0))],
        out_specs=[pl.BlockSpec((_BQ, d), lambda i: (i, 0))] * 3,
        out_shape=[jax.ShapeDtypeStruct((s, d), jnp.float32)] * 3,
    )(x2, Wq, Wc, Wk, Wv)
    q = q2.reshape(s, _H, _HD).transpose(1, 0, 2)
    k = k2.reshape(s, _H, _HD).transpose(1, 0, 2)
    v = v2.reshape(s, _H, _HD).transpose(1, 0, 2)
    rnd = _keep_mask(_H, s)
    attn_out = pl.pallas_call(
        functools.partial(_attn_body, bq=_BQ, s=s, scale=_HD ** -0.5),
        grid=(_H, s // _BQ),
        in_specs=[pl.BlockSpec((1, _BQ, _HD), lambda h, i: (h, i, 0)),
                  pl.BlockSpec((1, s, _HD), lambda h, i: (h, 0, 0)),
                  pl.BlockSpec((1, s, _HD), lambda h, i: (h, 0, 0)),
                  pl.BlockSpec((1, _BQ, s), lambda h, i: (h, i, 0))],
        out_specs=pl.BlockSpec((1, _BQ, _HD), lambda h, i: (h, i, 0)),
        out_shape=jax.ShapeDtypeStruct((_H, s, _HD), jnp.float32),
    )(q, k, v, rnd)
    a = attn_out.transpose(1, 0, 2).reshape(s, d)
    out = pl.pallas_call(
        _outproj_body,
        grid=(s // _BQ,),
        in_specs=[pl.BlockSpec((_BQ, d), lambda i: (i, 0)),
                  pl.BlockSpec((d, d), lambda i: (0, 0))],
        out_specs=pl.BlockSpec((_BQ, d), lambda i: (i, 0)),
        out_shape=jax.ShapeDtypeStruct((s, d), jnp.float32),
    )(a, Wo)
    return out.reshape(b, s, d)
